# baseline (speedup 1.0000x reference)
import functools

import jax
import jax.numpy as jnp
from jax import lax
from jax.experimental import pallas as pl
from jax.experimental.pallas import tpu as pltpu
from jax.experimental.pallas import tpu_sc as plsc

_NC = 2
_NS = 16
_NW = _NC * _NS

DIM = 1024
HIDDEN = 2816
NUM_EXPERTS = 8
TOP_K = 2
TM = 256

_INTERPRET = False


def _router_kernel(x_ref, rw_ref, w_ref, e_ref):
    x = x_ref[...]
    rw = rw_ref[...]
    logits = jax.lax.dot_general(
        x, rw, (((1,), (1,)), ((), ())), preferred_element_type=jnp.float32)
    m = jnp.max(logits, axis=-1, keepdims=True)
    ex = jnp.exp(logits - m)
    probs = ex / jnp.sum(ex, axis=-1, keepdims=True)

    lane = jax.lax.broadcasted_iota(jnp.int32, probs.shape, 1)
    w1 = jnp.max(probs, axis=-1, keepdims=True)
    e1 = jnp.min(jnp.where(probs == w1, lane, NUM_EXPERTS), axis=-1,
                 keepdims=True)
    masked = jnp.where(lane == e1, -1.0, probs)
    w2 = jnp.max(masked, axis=-1, keepdims=True)
    e2 = jnp.min(jnp.where(masked == w2, lane, NUM_EXPERTS), axis=-1,
                 keepdims=True)
    s = w1 + w2
    w_ref[...] = jnp.concatenate([w1 / s, w2 / s], axis=-1)
    e_ref[...] = jnp.concatenate([e1, e2], axis=-1)


def _route(x, router_w):
    T = x.shape[0]
    return pl.pallas_call(
        _router_kernel,
        out_shape=(
            jax.ShapeDtypeStruct((T, TOP_K), jnp.float32),
            jax.ShapeDtypeStruct((T, TOP_K), jnp.int32),
        ),
        interpret=_INTERPRET,
    )(x, router_w)


NH = 2
HB = HIDDEN // NH


def _mlp_kernel(te_ref, xg_ref, gate_ref, up_ref, down_ref, out_ref):
    del te_ref
    x = xg_ref[...]
    g = jax.lax.dot_general(
        x, gate_ref[0].astype(jnp.bfloat16), (((1,), (1,)), ((), ())),
        preferred_element_type=jnp.float32)
    u = jax.lax.dot_general(
        x, up_ref[0].astype(jnp.bfloat16), (((1,), (1,)), ((), ())),
        preferred_element_type=jnp.float32)
    h = (g * jax.lax.logistic(g)) * u
    out_ref[...] = jax.lax.dot_general(
        h.astype(jnp.bfloat16), down_ref[0].astype(jnp.bfloat16),
        (((1,), (1,)), ((), ())),
        preferred_element_type=jnp.float32)


def _grouped_mlp_half(xg, tile_expert, gate_w, up_w, down_w, h_idx):
    R = xg.shape[0]
    n_tiles = R // TM
    grid_spec = pltpu.PrefetchScalarGridSpec(
        num_scalar_prefetch=1,
        grid=(n_tiles,),
        in_specs=[
            pl.BlockSpec((TM, DIM), lambda i, te: (i, 0)),
            pl.BlockSpec((1, HB, DIM), lambda i, te: (te[i], h_idx, 0)),
            pl.BlockSpec((1, HB, DIM), lambda i, te: (te[i], h_idx, 0)),
            pl.BlockSpec((1, DIM, HB), lambda i, te: (te[i], 0, h_idx)),
        ],
        out_specs=pl.BlockSpec((TM, DIM), lambda i, te: (i, 0)),
    )
    return pl.pallas_call(
        _mlp_kernel,
        grid_spec=grid_spec,
        out_shape=jax.ShapeDtypeStruct((R, DIM), jnp.float32),
        interpret=_INTERPRET,
    )(tile_expert, xg, gate_w, up_w, down_w)


def _sc_gather(x_bf, gidx):
    T = x_bf.shape[0]
    R = gidx.shape[0]
    rpw = R // _NW
    CH = rpw // 2
    D2 = DIM // 2
    x32 = jax.lax.bitcast_convert_type(
        x_bf.reshape(T, D2, 2), jnp.int32)
    mesh = plsc.VectorSubcoreMesh(core_axis_name="c", subcore_axis_name="s")

    @functools.partial(
        pl.kernel, mesh=mesh,
        out_type=jax.ShapeDtypeStruct((R, D2), jnp.int32),
        scratch_types=[
            pltpu.VMEM((CH,), jnp.int32),
            pltpu.VMEM((CH, D2), jnp.int32),
            pltpu.SemaphoreType.DMA,
        ],
    )
    def k(x_hbm, gidx_hbm, out_hbm, idx_v, rows_v, sem):
        wid = lax.axis_index("s") * _NC + lax.axis_index("c")
        base = wid * rpw
        for c in range(2):
            pltpu.sync_copy(gidx_hbm.at[pl.ds(base + c * CH, CH)], idx_v)
            pltpu.async_copy(x_hbm.at[idx_v], rows_v, sem).wait()
            pltpu.sync_copy(rows_v, out_hbm.at[pl.ds(base + c * CH, CH)])

    out32 = k(x32, gidx)
    return jax.lax.bitcast_convert_type(
        out32, jnp.bfloat16).reshape(R, DIM)


def _sc_combine(y0, y1, s1, s2, w1, w2):
    T = s1.shape[0]
    tpw = T // _NW
    CH = 16
    NCH = tpw // CH
    mesh = plsc.VectorSubcoreMesh(core_axis_name="c", subcore_axis_name="s")

    @functools.partial(
        pl.kernel, mesh=mesh,
        out_type=jax.ShapeDtypeStruct((T, DIM), jnp.float32),
        scratch_types=[
            pltpu.VMEM((CH,), jnp.int32),
            pltpu.VMEM((CH,), jnp.int32),
            pltpu.VMEM((CH, 16), jnp.float32),
            pltpu.VMEM((CH, 16), jnp.float32),
            pltpu.VMEM((CH, DIM), jnp.float32),
            pltpu.VMEM((CH, DIM), jnp.float32),
            pltpu.VMEM((CH, DIM), jnp.float32),
            pltpu.VMEM((CH, DIM), jnp.float32),
            pltpu.VMEM((CH, DIM), jnp.float32),
            pltpu.SemaphoreType.DMA,
        ],
    )
    def k(y0_hbm, y1_hbm, s1_hbm, s2_hbm, w1_hbm, w2_hbm, out_hbm,
          s1_v, s2_v, w1_v, w2_v, a0, a1, b0, b1, o_v, sem):
        wid = lax.axis_index("s") * _NC + lax.axis_index("c")
        base = wid * tpw
        for c in range(NCH):
            cb = base + c * CH
            pltpu.sync_copy(s1_hbm.at[pl.ds(cb, CH)], s1_v)
            pltpu.sync_copy(s2_hbm.at[pl.ds(cb, CH)], s2_v)
            pltpu.sync_copy(w1_hbm.at[pl.ds(cb, CH)], w1_v)
            pltpu.sync_copy(w2_hbm.at[pl.ds(cb, CH)], w2_v)
            c0 = pltpu.async_copy(y0_hbm.at[s1_v], a0, sem)
            c1 = pltpu.async_copy(y1_hbm.at[s1_v], a1, sem)
            c2 = pltpu.async_copy(y0_hbm.at[s2_v], b0, sem)
            c3 = pltpu.async_copy(y1_hbm.at[s2_v], b1, sem)
            c0.wait(); c1.wait(); c2.wait(); c3.wait()
            for t in range(CH):
                w1s = w1_v[t, :]
                w2s = w2_v[t, :]

                def body(v, carry, t=t, w1s=w1s, w2s=w2s):
                    sl = pl.ds(v * 16, 16)
                    o_v[t, sl] = (w1s * (a0[t, sl] + a1[t, sl])
                                  + w2s * (b0[t, sl] + b1[t, sl]))
                    return carry

                lax.fori_loop(0, DIM // 16, body, 0)
            pltpu.sync_copy(o_v, out_hbm.at[pl.ds(cb, CH)])

    return k(y0, y1, s1, s2, w1, w2)


def kernel(hidden_states, router_w, gate_w, up_w, down_w):
    B, S, D = hidden_states.shape
    T = B * S
    x = hidden_states.reshape(T, D)

    w, e = _route(x, router_w)

    e_flat = e.reshape(-1)
    oh = (e_flat[:, None] == jnp.arange(NUM_EXPERTS)[None, :]).astype(jnp.int32)
    pos = jnp.cumsum(oh, axis=0) - oh
    pos_i = jnp.sum(pos * oh, axis=-1)
    counts = jnp.sum(oh, axis=0)
    padded = ((counts + TM - 1) // TM) * TM
    ends = jnp.cumsum(padded)
    offs = ends - padded
    slot = offs[e_flat] + pos_i

    R = T * TOP_K + NUM_EXPERTS * TM
    gidx = jnp.zeros((R,), jnp.int32).at[slot].set(
        jnp.arange(T * TOP_K, dtype=jnp.int32) // TOP_K)
    n_tiles = R // TM
    tile_expert = jnp.minimum(
        jnp.sum(jnp.arange(n_tiles)[:, None] >= (ends // TM)[None, :], axis=-1),
        NUM_EXPERTS - 1).astype(jnp.int32)

    xg = _sc_gather(x.astype(jnp.bfloat16), gidx)
    y0 = _grouped_mlp_half(xg, tile_expert, gate_w, up_w, down_w, 0)
    y1 = _grouped_mlp_half(xg, tile_expert, gate_w, up_w, down_w, 1)

    s_tok = slot.reshape(T, TOP_K)
    out = _sc_combine(y0, y1,
                      s_tok[:, 0] + 0,
                      s_tok[:, 1] + 0,
                      jnp.broadcast_to(w[:, 0:1], (T, 16)) + 0.0,
                      jnp.broadcast_to(w[:, 1:2], (T, 16)) + 0.0)
    return out.reshape(B, S, D)


# speedup vs baseline: 1.0249x; 1.0249x over previous
import functools

import jax
import jax.numpy as jnp
from jax import lax
from jax.experimental import pallas as pl
from jax.experimental.pallas import tpu as pltpu
from jax.experimental.pallas import tpu_sc as plsc

_NC = 2
_NS = 16
_NW = _NC * _NS

DIM = 1024
HIDDEN = 2816
NUM_EXPERTS = 8
TOP_K = 2
TM = 256

_INTERPRET = False


def _router_kernel(x_ref, rw_ref, w_ref, e_ref):
    x = x_ref[...]
    rw = rw_ref[...]
    logits = jax.lax.dot_general(
        x, rw, (((1,), (1,)), ((), ())), preferred_element_type=jnp.float32)
    m = jnp.max(logits, axis=-1, keepdims=True)
    ex = jnp.exp(logits - m)
    probs = ex / jnp.sum(ex, axis=-1, keepdims=True)

    lane = jax.lax.broadcasted_iota(jnp.int32, probs.shape, 1)
    w1 = jnp.max(probs, axis=-1, keepdims=True)
    e1 = jnp.min(jnp.where(probs == w1, lane, NUM_EXPERTS), axis=-1,
                 keepdims=True)
    masked = jnp.where(lane == e1, -1.0, probs)
    w2 = jnp.max(masked, axis=-1, keepdims=True)
    e2 = jnp.min(jnp.where(masked == w2, lane, NUM_EXPERTS), axis=-1,
                 keepdims=True)
    s = w1 + w2
    w_ref[...] = jnp.concatenate([w1 / s, w2 / s], axis=-1)
    e_ref[...] = jnp.concatenate([e1, e2], axis=-1)


def _route(x, router_w):
    T = x.shape[0]
    return pl.pallas_call(
        _router_kernel,
        out_shape=(
            jax.ShapeDtypeStruct((T, TOP_K), jnp.float32),
            jax.ShapeDtypeStruct((T, TOP_K), jnp.int32),
        ),
        interpret=_INTERPRET,
    )(x, router_w)


NH = 2
HB = HIDDEN // NH


def _mlp_kernel(te_ref, xg_ref, gate_ref, up_ref, down_ref, out_ref):
    del te_ref
    x = xg_ref[...]
    g = jax.lax.dot_general(
        x, gate_ref[0].astype(jnp.bfloat16), (((1,), (1,)), ((), ())),
        preferred_element_type=jnp.float32)
    u = jax.lax.dot_general(
        x, up_ref[0].astype(jnp.bfloat16), (((1,), (1,)), ((), ())),
        preferred_element_type=jnp.float32)
    h = (g * jax.lax.logistic(g)) * u
    out_ref[...] = jax.lax.dot_general(
        h.astype(jnp.bfloat16), down_ref[0].astype(jnp.bfloat16),
        (((1,), (1,)), ((), ())),
        preferred_element_type=jnp.float32)


def _grouped_mlp_half(xg, tile_expert, gate_w, up_w, down_w, h_idx):
    R = xg.shape[0]
    n_tiles = R // TM
    grid_spec = pltpu.PrefetchScalarGridSpec(
        num_scalar_prefetch=1,
        grid=(n_tiles,),
        in_specs=[
            pl.BlockSpec((TM, DIM), lambda i, te: (i, 0)),
            pl.BlockSpec((1, HB, DIM), lambda i, te: (te[i], h_idx, 0)),
            pl.BlockSpec((1, HB, DIM), lambda i, te: (te[i], h_idx, 0)),
            pl.BlockSpec((1, DIM, HB), lambda i, te: (te[i], 0, h_idx)),
        ],
        out_specs=pl.BlockSpec((TM, DIM), lambda i, te: (i, 0)),
    )
    return pl.pallas_call(
        _mlp_kernel,
        grid_spec=grid_spec,
        out_shape=jax.ShapeDtypeStruct((R, DIM), jnp.float32),
        interpret=_INTERPRET,
    )(tile_expert, xg, gate_w, up_w, down_w)


def _sc_gather(x_bf, gidx):
    T = x_bf.shape[0]
    R = gidx.shape[0]
    rpw = R // _NW
    CH = rpw // 2
    D2 = DIM // 2
    x32 = jax.lax.bitcast_convert_type(
        x_bf.reshape(T, D2, 2), jnp.int32)
    mesh = plsc.VectorSubcoreMesh(core_axis_name="c", subcore_axis_name="s")

    @functools.partial(
        pl.kernel, mesh=mesh,
        out_type=jax.ShapeDtypeStruct((R, D2), jnp.int32),
        scratch_types=[
            pltpu.VMEM((CH,), jnp.int32),
            pltpu.VMEM((CH, D2), jnp.int32),
            pltpu.SemaphoreType.DMA,
        ],
    )
    def k(x_hbm, gidx_hbm, out_hbm, idx_v, rows_v, sem):
        wid = lax.axis_index("s") * _NC + lax.axis_index("c")
        base = wid * rpw
        for c in range(2):
            pltpu.sync_copy(gidx_hbm.at[pl.ds(base + c * CH, CH)], idx_v)
            pltpu.async_copy(x_hbm.at[idx_v], rows_v, sem).wait()
            pltpu.sync_copy(rows_v, out_hbm.at[pl.ds(base + c * CH, CH)])

    out32 = k(x32, gidx)
    return jax.lax.bitcast_convert_type(
        out32, jnp.bfloat16).reshape(R, DIM)


def _sc_combine(y0, y1, s1, s2, w1, w2):
    T = s1.shape[0]
    tpw = T // _NW
    CH = 16
    NCH = tpw // CH
    mesh = plsc.VectorSubcoreMesh(core_axis_name="c", subcore_axis_name="s")

    @functools.partial(
        pl.kernel, mesh=mesh,
        out_type=jax.ShapeDtypeStruct((T, DIM), jnp.float32),
        scratch_types=[
            pltpu.VMEM((CH,), jnp.int32),
            pltpu.VMEM((CH,), jnp.int32),
            pltpu.VMEM((CH, 16), jnp.float32),
            pltpu.VMEM((CH, 16), jnp.float32),
            pltpu.VMEM((CH, DIM), jnp.float32),
            pltpu.VMEM((CH, DIM), jnp.float32),
            pltpu.VMEM((CH, DIM), jnp.float32),
            pltpu.VMEM((CH, DIM), jnp.float32),
            pltpu.VMEM((CH, DIM), jnp.float32),
            pltpu.SemaphoreType.DMA,
        ],
    )
    def k(y0_hbm, y1_hbm, s1_hbm, s2_hbm, w1_hbm, w2_hbm, out_hbm,
          s1_v, s2_v, w1_v, w2_v, a0, a1, b0, b1, o_v, sem):
        wid = lax.axis_index("s") * _NC + lax.axis_index("c")
        base = wid * tpw
        for c in range(NCH):
            cb = base + c * CH
            pltpu.sync_copy(s1_hbm.at[pl.ds(cb, CH)], s1_v)
            pltpu.sync_copy(s2_hbm.at[pl.ds(cb, CH)], s2_v)
            pltpu.sync_copy(w1_hbm.at[pl.ds(cb, CH)], w1_v)
            pltpu.sync_copy(w2_hbm.at[pl.ds(cb, CH)], w2_v)
            c0 = pltpu.async_copy(y0_hbm.at[s1_v], a0, sem)
            c1 = pltpu.async_copy(y1_hbm.at[s1_v], a1, sem)
            c2 = pltpu.async_copy(y0_hbm.at[s2_v], b0, sem)
            c3 = pltpu.async_copy(y1_hbm.at[s2_v], b1, sem)
            c0.wait(); c1.wait(); c2.wait(); c3.wait()
            for t in range(CH):
                w1s = w1_v[t, :]
                w2s = w2_v[t, :]

                def body(v, carry, t=t, w1s=w1s, w2s=w2s):
                    sl = pl.ds(v * 16, 16)
                    o_v[t, sl] = (w1s * (a0[t, sl] + a1[t, sl])
                                  + w2s * (b0[t, sl] + b1[t, sl]))
                    return carry

                lax.fori_loop(0, DIM // 16, body, 0)
            pltpu.sync_copy(o_v, out_hbm.at[pl.ds(cb, CH)])

    return k(y0, y1, s1, s2, w1, w2)


def kernel(hidden_states, router_w, gate_w, up_w, down_w):
    B, S, D = hidden_states.shape
    T = B * S
    x = hidden_states.reshape(T, D)

    w, e = _route(x, router_w)

    e_flat = e.reshape(-1)
    oh = (e_flat[:, None] == jnp.arange(NUM_EXPERTS)[None, :]).astype(jnp.int32)
    pos = jnp.cumsum(oh, axis=0) - oh
    pos_i = jnp.sum(pos * oh, axis=-1)
    counts = jnp.sum(oh, axis=0)
    padded = ((counts + TM - 1) // TM) * TM
    ends = jnp.cumsum(padded)
    offs = ends - padded
    slot = offs[e_flat] + pos_i

    R = T * TOP_K + NUM_EXPERTS * TM
    gidx = jnp.zeros((R,), jnp.int32).at[slot].set(
        jnp.arange(T * TOP_K, dtype=jnp.int32) // TOP_K)
    n_tiles = R // TM
    tile_expert = jnp.minimum(
        jnp.sum(jnp.arange(n_tiles)[:, None] >= (ends // TM)[None, :], axis=-1),
        NUM_EXPERTS - 1).astype(jnp.int32)

    xg = _sc_gather(x.astype(jnp.bfloat16), gidx)
    y0 = _grouped_mlp_half(xg, tile_expert, gate_w, up_w, down_w, 0)
    y1 = _grouped_mlp_half(xg, tile_expert, gate_w, up_w, down_w, 1)

    s_tok = slot.reshape(T, TOP_K)
    out = (w[:, 0:1] * (jnp.take(y0, s_tok[:, 0], axis=0)
                        + jnp.take(y1, s_tok[:, 0], axis=0))
           + w[:, 1:2] * (jnp.take(y0, s_tok[:, 1], axis=0)
                          + jnp.take(y1, s_tok[:, 1], axis=0)))
    return out.reshape(B, S, D)


# speedup vs baseline: 1.5241x; 1.4872x over previous
import functools

import jax
import jax.numpy as jnp
from jax import lax
from jax.experimental import pallas as pl
from jax.experimental.pallas import tpu as pltpu
from jax.experimental.pallas import tpu_sc as plsc

_NC = 2
_NS = 16
_NW = _NC * _NS

DIM = 1024
HIDDEN = 2816
NUM_EXPERTS = 8
TOP_K = 2
TM = 256

_INTERPRET = False


def _router_kernel(x_ref, rw_ref, w_ref, e_ref):
    x = x_ref[...]
    rw = rw_ref[...]
    logits = jax.lax.dot_general(
        x, rw, (((1,), (1,)), ((), ())), preferred_element_type=jnp.float32)
    m = jnp.max(logits, axis=-1, keepdims=True)
    ex = jnp.exp(logits - m)
    probs = ex / jnp.sum(ex, axis=-1, keepdims=True)

    lane = jax.lax.broadcasted_iota(jnp.int32, probs.shape, 1)
    w1 = jnp.max(probs, axis=-1, keepdims=True)
    e1 = jnp.min(jnp.where(probs == w1, lane, NUM_EXPERTS), axis=-1,
                 keepdims=True)
    masked = jnp.where(lane == e1, -1.0, probs)
    w2 = jnp.max(masked, axis=-1, keepdims=True)
    e2 = jnp.min(jnp.where(masked == w2, lane, NUM_EXPERTS), axis=-1,
                 keepdims=True)
    s = w1 + w2
    w_ref[...] = jnp.concatenate([w1 / s, w2 / s], axis=-1)
    e_ref[...] = jnp.concatenate([e1, e2], axis=-1)


def _route(x, router_w):
    T = x.shape[0]
    return pl.pallas_call(
        _router_kernel,
        out_shape=(
            jax.ShapeDtypeStruct((T, TOP_K), jnp.float32),
            jax.ShapeDtypeStruct((T, TOP_K), jnp.int32),
        ),
        interpret=_INTERPRET,
    )(x, router_w)


NH = 2
HB = HIDDEN // NH


def _mlp_kernel(te_ref, xg_ref, gate_ref, up_ref, down_ref, out_ref):
    del te_ref
    x = xg_ref[...]
    g = jax.lax.dot_general(
        x, gate_ref[0].astype(jnp.bfloat16), (((1,), (1,)), ((), ())),
        preferred_element_type=jnp.float32)
    u = jax.lax.dot_general(
        x, up_ref[0].astype(jnp.bfloat16), (((1,), (1,)), ((), ())),
        preferred_element_type=jnp.float32)
    h = (g * jax.lax.logistic(g)) * u
    out_ref[...] = jax.lax.dot_general(
        h.astype(jnp.bfloat16), down_ref[0].astype(jnp.bfloat16),
        (((1,), (1,)), ((), ())),
        preferred_element_type=jnp.float32)


def _grouped_mlp(xg, tile_expert, gate_w, up_w, down_w):
    R = xg.shape[0]
    n_tiles = R // TM
    grid_spec = pltpu.PrefetchScalarGridSpec(
        num_scalar_prefetch=1,
        grid=(NH, n_tiles),
        in_specs=[
            pl.BlockSpec((TM, DIM), lambda h, i, te: (i, 0)),
            pl.BlockSpec((1, HB, DIM), lambda h, i, te: (te[i], h, 0)),
            pl.BlockSpec((1, HB, DIM), lambda h, i, te: (te[i], h, 0)),
            pl.BlockSpec((1, DIM, HB), lambda h, i, te: (te[i], 0, h)),
        ],
        out_specs=pl.BlockSpec((TM, DIM), lambda h, i, te: (i, h)),
    )
    return pl.pallas_call(
        _mlp_kernel,
        grid_spec=grid_spec,
        out_shape=jax.ShapeDtypeStruct((R, NH * DIM), jnp.float32),
        interpret=_INTERPRET,
    )(tile_expert, xg, gate_w, up_w, down_w)


def _sc_gather(x_bf, gidx):
    T = x_bf.shape[0]
    R = gidx.shape[0]
    rpw = R // _NW
    CH = rpw // 2
    D2 = DIM // 2
    x32 = jax.lax.bitcast_convert_type(
        x_bf.reshape(T, D2, 2), jnp.int32)
    mesh = plsc.VectorSubcoreMesh(core_axis_name="c", subcore_axis_name="s")

    @functools.partial(
        pl.kernel, mesh=mesh,
        out_type=jax.ShapeDtypeStruct((R, D2), jnp.int32),
        scratch_types=[
            pltpu.VMEM((CH,), jnp.int32),
            pltpu.VMEM((CH, D2), jnp.int32),
            pltpu.SemaphoreType.DMA,
        ],
    )
    def k(x_hbm, gidx_hbm, out_hbm, idx_v, rows_v, sem):
        wid = lax.axis_index("s") * _NC + lax.axis_index("c")
        base = wid * rpw
        for c in range(2):
            pltpu.sync_copy(gidx_hbm.at[pl.ds(base + c * CH, CH)], idx_v)
            pltpu.async_copy(x_hbm.at[idx_v], rows_v, sem).wait()
            pltpu.sync_copy(rows_v, out_hbm.at[pl.ds(base + c * CH, CH)])

    out32 = k(x32, gidx)
    return jax.lax.bitcast_convert_type(
        out32, jnp.bfloat16).reshape(R, DIM)


def _sc_combine(y0, y1, s1, s2, w1, w2):
    T = s1.shape[0]
    tpw = T // _NW
    CH = 16
    NCH = tpw // CH
    mesh = plsc.VectorSubcoreMesh(core_axis_name="c", subcore_axis_name="s")

    @functools.partial(
        pl.kernel, mesh=mesh,
        out_type=jax.ShapeDtypeStruct((T, DIM), jnp.float32),
        scratch_types=[
            pltpu.VMEM((CH,), jnp.int32),
            pltpu.VMEM((CH,), jnp.int32),
            pltpu.VMEM((CH,), jnp.float32),
            pltpu.VMEM((CH,), jnp.float32),
            pltpu.VMEM((CH, DIM), jnp.float32),
            pltpu.VMEM((CH, DIM), jnp.float32),
            pltpu.VMEM((CH, DIM), jnp.float32),
            pltpu.VMEM((CH, DIM), jnp.float32),
            pltpu.VMEM((CH, DIM), jnp.float32),
            pltpu.SemaphoreType.DMA,
        ],
    )
    def k(y0_hbm, y1_hbm, s1_hbm, s2_hbm, w1_hbm, w2_hbm, out_hbm,
          s1_v, s2_v, w1_v, w2_v, a0, a1, b0, b1, o_v, sem):
        wid = lax.axis_index("s") * _NC + lax.axis_index("c")
        base = wid * tpw
        for c in range(NCH):
            cb = base + c * CH
            pltpu.sync_copy(s1_hbm.at[pl.ds(cb, CH)], s1_v)
            pltpu.sync_copy(s2_hbm.at[pl.ds(cb, CH)], s2_v)
            pltpu.sync_copy(w1_hbm.at[pl.ds(cb, CH)], w1_v)
            pltpu.sync_copy(w2_hbm.at[pl.ds(cb, CH)], w2_v)
            c0 = pltpu.async_copy(y0_hbm.at[s1_v], a0, sem)
            c1 = pltpu.async_copy(y1_hbm.at[s1_v], a1, sem)
            c2 = pltpu.async_copy(y0_hbm.at[s2_v], b0, sem)
            c3 = pltpu.async_copy(y1_hbm.at[s2_v], b1, sem)
            c0.wait(); c1.wait(); c2.wait(); c3.wait()
            for t in range(CH):
                tid = lax.full((16,), t, jnp.int32)
                w1s = plsc.load_gather(w1_v, [tid])
                w2s = plsc.load_gather(w2_v, [tid])

                def body(v, carry, t=t, w1s=w1s, w2s=w2s):
                    sl = pl.ds(v * 16, 16)
                    o_v[t, sl] = (w1s * (a0[t, sl] + a1[t, sl])
                                  + w2s * (b0[t, sl] + b1[t, sl]))
                    return carry

                lax.fori_loop(0, DIM // 16, body, 0)
            pltpu.sync_copy(o_v, out_hbm.at[pl.ds(cb, CH)])

    return k(y0, y1, s1, s2, w1, w2)


def kernel(hidden_states, router_w, gate_w, up_w, down_w):
    B, S, D = hidden_states.shape
    T = B * S
    x = hidden_states.reshape(T, D)

    w, e = _route(x, router_w)

    e_flat = e.reshape(-1)
    oh = (e_flat[:, None] == jnp.arange(NUM_EXPERTS)[None, :]).astype(jnp.int32)
    pos = jnp.cumsum(oh, axis=0) - oh
    pos_i = jnp.sum(pos * oh, axis=-1)
    counts = jnp.sum(oh, axis=0)
    padded = ((counts + TM - 1) // TM) * TM
    ends = jnp.cumsum(padded)
    offs = ends - padded
    slot = offs[e_flat] + pos_i

    R = T * TOP_K + NUM_EXPERTS * TM
    gidx = jnp.zeros((R,), jnp.int32).at[slot].set(
        jnp.arange(T * TOP_K, dtype=jnp.int32) // TOP_K)
    n_tiles = R // TM
    tile_expert = jnp.minimum(
        jnp.sum(jnp.arange(n_tiles)[:, None] >= (ends // TM)[None, :], axis=-1),
        NUM_EXPERTS - 1).astype(jnp.int32)

    xg = jnp.take(x.astype(jnp.bfloat16), gidx, axis=0)
    y = _grouped_mlp(xg, tile_expert, gate_w, up_w, down_w)

    s_tok = slot.reshape(T, TOP_K)
    ya = jnp.take(y, s_tok[:, 0], axis=0)
    yb = jnp.take(y, s_tok[:, 1], axis=0)
    out = (w[:, 0:1] * (ya[:, :DIM] + ya[:, DIM:])
           + w[:, 1:2] * (yb[:, :DIM] + yb[:, DIM:]))
    return out.reshape(B, S, D)


# speedup vs baseline: 1.6391x; 1.0754x over previous
import functools

import jax
import jax.numpy as jnp
from jax import lax
from jax.experimental import pallas as pl
from jax.experimental.pallas import tpu as pltpu
from jax.experimental.pallas import tpu_sc as plsc

_NC = 2
_NS = 16
_NW = _NC * _NS

DIM = 1024
HIDDEN = 2816
NUM_EXPERTS = 8
TOP_K = 2
TM = 256

_INTERPRET = False


def _router_kernel(x_ref, rw_ref, w_ref, e_ref):
    x = x_ref[...]
    rw = rw_ref[...]
    logits = jax.lax.dot_general(
        x, rw, (((1,), (1,)), ((), ())), preferred_element_type=jnp.float32)
    m = jnp.max(logits, axis=-1, keepdims=True)
    ex = jnp.exp(logits - m)
    probs = ex / jnp.sum(ex, axis=-1, keepdims=True)

    lane = jax.lax.broadcasted_iota(jnp.int32, probs.shape, 1)
    w1 = jnp.max(probs, axis=-1, keepdims=True)
    e1 = jnp.min(jnp.where(probs == w1, lane, NUM_EXPERTS), axis=-1,
                 keepdims=True)
    masked = jnp.where(lane == e1, -1.0, probs)
    w2 = jnp.max(masked, axis=-1, keepdims=True)
    e2 = jnp.min(jnp.where(masked == w2, lane, NUM_EXPERTS), axis=-1,
                 keepdims=True)
    s = w1 + w2
    w_ref[...] = jnp.concatenate([w1 / s, w2 / s], axis=-1)
    e_ref[...] = jnp.concatenate([e1, e2], axis=-1)


def _route(x, router_w):
    T = x.shape[0]
    return pl.pallas_call(
        _router_kernel,
        out_shape=(
            jax.ShapeDtypeStruct((T, TOP_K), jnp.float32),
            jax.ShapeDtypeStruct((T, TOP_K), jnp.int32),
        ),
        interpret=_INTERPRET,
    )(x, router_w)


NH = 2
HB = HIDDEN // NH


def _mlp_kernel(te_ref, xg_ref, gate_ref, up_ref, down_ref, out_ref):
    del te_ref
    x = xg_ref[...]
    g = jax.lax.dot_general(
        x, gate_ref[0].astype(jnp.bfloat16), (((1,), (1,)), ((), ())),
        preferred_element_type=jnp.float32)
    u = jax.lax.dot_general(
        x, up_ref[0].astype(jnp.bfloat16), (((1,), (1,)), ((), ())),
        preferred_element_type=jnp.float32)
    h = (g * jax.lax.logistic(g)) * u
    out_ref[...] = jax.lax.dot_general(
        h.astype(jnp.bfloat16), down_ref[0].astype(jnp.bfloat16),
        (((1,), (1,)), ((), ())),
        preferred_element_type=jnp.float32)


def _grouped_mlp_half(xg, tile_expert, gate_w, up_w, down_w, h_idx):
    R = xg.shape[0]
    n_tiles = R // TM
    grid_spec = pltpu.PrefetchScalarGridSpec(
        num_scalar_prefetch=1,
        grid=(n_tiles,),
        in_specs=[
            pl.BlockSpec((TM, DIM), lambda i, te: (i, 0)),
            pl.BlockSpec((1, HB, DIM), lambda i, te: (te[i], h_idx, 0)),
            pl.BlockSpec((1, HB, DIM), lambda i, te: (te[i], h_idx, 0)),
            pl.BlockSpec((1, DIM, HB), lambda i, te: (te[i], 0, h_idx)),
        ],
        out_specs=pl.BlockSpec((TM, DIM), lambda i, te: (i, 0)),
    )
    return pl.pallas_call(
        _mlp_kernel,
        grid_spec=grid_spec,
        out_shape=jax.ShapeDtypeStruct((R, DIM), jnp.float32),
        interpret=_INTERPRET,
    )(tile_expert, xg, gate_w, up_w, down_w)


def _sc_combine(y0, y1, s1_2d, s2_2d, w1e, w2e):
    T = w1e.shape[0]
    tpw = T // _NW
    CH = 16
    NCH = tpw // CH
    mesh = plsc.VectorSubcoreMesh(core_axis_name="c", subcore_axis_name="s")

    @functools.partial(
        pl.kernel, mesh=mesh,
        out_type=jax.ShapeDtypeStruct((T, DIM), jnp.float32),
        scratch_types=[
            pltpu.VMEM((NCH, CH), jnp.int32),
            pltpu.VMEM((NCH, CH), jnp.int32),
            pltpu.VMEM((tpw, 16), jnp.float32),
            pltpu.VMEM((tpw, 16), jnp.float32),
            pltpu.VMEM((CH, DIM), jnp.float32),
            pltpu.VMEM((CH, DIM), jnp.float32),
            pltpu.VMEM((CH, DIM), jnp.float32),
            pltpu.VMEM((CH, DIM), jnp.float32),
            pltpu.VMEM((CH, DIM), jnp.float32),
            pltpu.SemaphoreType.DMA,
        ],
    )
    def k(y0_hbm, y1_hbm, s1_hbm, s2_hbm, w1_hbm, w2_hbm, out_hbm,
          s1_v, s2_v, w1_v, w2_v, a0, a1, b0, b1, o_v, sem):
        wid = lax.axis_index("s") * _NC + lax.axis_index("c")
        rowbase = wid * NCH
        tokbase = wid * tpw
        pltpu.sync_copy(s1_hbm.at[pl.ds(rowbase, NCH)], s1_v)
        pltpu.sync_copy(s2_hbm.at[pl.ds(rowbase, NCH)], s2_v)
        pltpu.sync_copy(w1_hbm.at[pl.ds(tokbase, tpw)], w1_v)
        pltpu.sync_copy(w2_hbm.at[pl.ds(tokbase, tpw)], w2_v)
        for c in range(NCH):
            c0 = pltpu.async_copy(y0_hbm.at[s1_v.at[c]], a0, sem)
            c1 = pltpu.async_copy(y1_hbm.at[s1_v.at[c]], a1, sem)
            c2 = pltpu.async_copy(y0_hbm.at[s2_v.at[c]], b0, sem)
            c3 = pltpu.async_copy(y1_hbm.at[s2_v.at[c]], b1, sem)
            c0.wait(); c1.wait(); c2.wait(); c3.wait()
            for t in range(CH):
                w1s = w1_v[c * CH + t, :]
                w2s = w2_v[c * CH + t, :]

                def body(v, carry, t=t, w1s=w1s, w2s=w2s):
                    sl = pl.ds(v * 16, 16)
                    o_v[t, sl] = (w1s * (a0[t, sl] + a1[t, sl])
                                  + w2s * (b0[t, sl] + b1[t, sl]))
                    return carry

                lax.fori_loop(0, DIM // 16, body, 0)
            pltpu.sync_copy(o_v, out_hbm.at[pl.ds(tokbase + c * CH, CH)])

    return k(y0, y1, s1_2d, s2_2d, w1e, w2e)


def kernel(hidden_states, router_w, gate_w, up_w, down_w):
    B, S, D = hidden_states.shape
    T = B * S
    x = hidden_states.reshape(T, D)

    w, e = _route(x, router_w)

    e_flat = e.reshape(-1)
    oh = (e_flat[:, None] == jnp.arange(NUM_EXPERTS)[None, :]).astype(jnp.int32)
    pos = jnp.cumsum(oh, axis=0) - oh
    pos_i = jnp.sum(pos * oh, axis=-1)
    counts = jnp.sum(oh, axis=0)
    padded = ((counts + TM - 1) // TM) * TM
    ends = jnp.cumsum(padded)
    offs = ends - padded
    slot = offs[e_flat] + pos_i

    R = T * TOP_K + NUM_EXPERTS * TM
    gidx = jnp.zeros((R,), jnp.int32).at[slot].set(
        jnp.arange(T * TOP_K, dtype=jnp.int32) // TOP_K)
    n_tiles = R // TM
    tile_expert = jnp.minimum(
        jnp.sum(jnp.arange(n_tiles)[:, None] >= (ends // TM)[None, :], axis=-1),
        NUM_EXPERTS - 1).astype(jnp.int32)

    xg = jnp.take(x.astype(jnp.bfloat16), gidx, axis=0)
    y0 = _grouped_mlp_half(xg, tile_expert, gate_w, up_w, down_w, 0)
    y1 = _grouped_mlp_half(xg, tile_expert, gate_w, up_w, down_w, 1)

    s_tok = slot.reshape(T, TOP_K)
    out = _sc_combine(
        y0, y1,
        (s_tok[:, 0] + 0).reshape(T // 16, 16),
        (s_tok[:, 1] + 0).reshape(T // 16, 16),
        jnp.broadcast_to(w[:, 0:1], (T, 16)) + 0.0,
        jnp.broadcast_to(w[:, 1:2], (T, 16)) + 0.0)
    return out.reshape(B, S, D)


# speedup vs baseline: 1.7148x; 1.0462x over previous
import functools

import jax
import jax.numpy as jnp
from jax import lax
from jax.experimental import pallas as pl
from jax.experimental.pallas import tpu as pltpu
from jax.experimental.pallas import tpu_sc as plsc

_NC = 2
_NS = 16
_NW = _NC * _NS

DIM = 1024
HIDDEN = 2816
NUM_EXPERTS = 8
TOP_K = 2
TM = 256

_INTERPRET = False


def _router_kernel(x_ref, rw_ref, w_ref, e_ref):
    x = x_ref[...]
    rw = rw_ref[...]
    logits = jax.lax.dot_general(
        x, rw, (((1,), (1,)), ((), ())), preferred_element_type=jnp.float32)
    m = jnp.max(logits, axis=-1, keepdims=True)
    ex = jnp.exp(logits - m)
    probs = ex / jnp.sum(ex, axis=-1, keepdims=True)

    lane = jax.lax.broadcasted_iota(jnp.int32, probs.shape, 1)
    w1 = jnp.max(probs, axis=-1, keepdims=True)
    e1 = jnp.min(jnp.where(probs == w1, lane, NUM_EXPERTS), axis=-1,
                 keepdims=True)
    masked = jnp.where(lane == e1, -1.0, probs)
    w2 = jnp.max(masked, axis=-1, keepdims=True)
    e2 = jnp.min(jnp.where(masked == w2, lane, NUM_EXPERTS), axis=-1,
                 keepdims=True)
    s = w1 + w2
    w_ref[...] = jnp.concatenate([w1 / s, w2 / s], axis=-1)
    e_ref[...] = jnp.concatenate([e1, e2], axis=-1)


def _route(x, router_w):
    T = x.shape[0]
    return pl.pallas_call(
        _router_kernel,
        out_shape=(
            jax.ShapeDtypeStruct((T, TOP_K), jnp.float32),
            jax.ShapeDtypeStruct((T, TOP_K), jnp.int32),
        ),
        interpret=_INTERPRET,
    )(x, router_w)


NH = 2
HB = HIDDEN // NH


def _mlp_kernel(te_ref, xg_ref, gate_ref, up_ref, down_ref, out_ref):
    del te_ref
    x = xg_ref[...]
    g = jax.lax.dot_general(
        x, gate_ref[0].astype(jnp.bfloat16), (((1,), (1,)), ((), ())),
        preferred_element_type=jnp.float32)
    u = jax.lax.dot_general(
        x, up_ref[0].astype(jnp.bfloat16), (((1,), (1,)), ((), ())),
        preferred_element_type=jnp.float32)
    h = (g * jax.lax.logistic(g)) * u
    out_ref[...] = jax.lax.dot_general(
        h.astype(jnp.bfloat16), down_ref[0].astype(jnp.bfloat16),
        (((1,), (1,)), ((), ())),
        preferred_element_type=jnp.float32)


def _grouped_mlp_half(xg, tile_expert, gate_w, up_w, down_w, h_idx):
    R = xg.shape[0]
    n_tiles = R // TM
    grid_spec = pltpu.PrefetchScalarGridSpec(
        num_scalar_prefetch=1,
        grid=(n_tiles,),
        in_specs=[
            pl.BlockSpec((TM, DIM), lambda i, te: (i, 0)),
            pl.BlockSpec((1, HB, DIM), lambda i, te: (te[i], h_idx, 0)),
            pl.BlockSpec((1, HB, DIM), lambda i, te: (te[i], h_idx, 0)),
            pl.BlockSpec((1, DIM, HB), lambda i, te: (te[i], 0, h_idx)),
        ],
        out_specs=pl.BlockSpec((TM, DIM), lambda i, te: (i, 0)),
    )
    return pl.pallas_call(
        _mlp_kernel,
        grid_spec=grid_spec,
        out_shape=jax.ShapeDtypeStruct((R, DIM), jnp.float32),
        interpret=_INTERPRET,
    )(tile_expert, xg, gate_w, up_w, down_w)


def _sc_combine(y0, y1, s1_2d, s2_2d, w1e, w2e):
    T = w1e.shape[0]
    tpw = T // _NW
    CH = 8
    NCH = tpw // CH
    mesh = plsc.VectorSubcoreMesh(core_axis_name="c", subcore_axis_name="s")

    row_buf = pltpu.VMEM((CH, DIM), jnp.float32)

    @functools.partial(
        pl.kernel, mesh=mesh,
        out_type=jax.ShapeDtypeStruct((T, DIM), jnp.float32),
        scratch_types=[
            pltpu.VMEM((NCH, CH), jnp.int32),
            pltpu.VMEM((NCH, CH), jnp.int32),
            pltpu.VMEM((tpw, 16), jnp.float32),
            pltpu.VMEM((tpw, 16), jnp.float32),
            row_buf, row_buf, row_buf, row_buf,
            row_buf, row_buf, row_buf, row_buf,
            pltpu.VMEM((CH, DIM), jnp.float32),
            pltpu.SemaphoreType.DMA,
            pltpu.SemaphoreType.DMA,
        ],
    )
    def k(y0_hbm, y1_hbm, s1_hbm, s2_hbm, w1_hbm, w2_hbm, out_hbm,
          s1_v, s2_v, w1_v, w2_v,
          a0A, a1A, b0A, b1A, a0B, a1B, b0B, b1B, o_v, semA, semB):
        wid = lax.axis_index("s") * _NC + lax.axis_index("c")
        rowbase = wid * NCH
        tokbase = wid * tpw
        pltpu.sync_copy(s1_hbm.at[pl.ds(rowbase, NCH)], s1_v)
        pltpu.sync_copy(s2_hbm.at[pl.ds(rowbase, NCH)], s2_v)
        pltpu.sync_copy(w1_hbm.at[pl.ds(tokbase, tpw)], w1_v)
        pltpu.sync_copy(w2_hbm.at[pl.ds(tokbase, tpw)], w2_v)
        bufs = ((a0A, a1A, b0A, b1A, semA), (a0B, a1B, b0B, b1B, semB))

        def issue(c):
            a0, a1, b0, b1, sem = bufs[c % 2]
            return (pltpu.async_copy(y0_hbm.at[s1_v.at[c]], a0, sem),
                    pltpu.async_copy(y1_hbm.at[s1_v.at[c]], a1, sem),
                    pltpu.async_copy(y0_hbm.at[s2_v.at[c]], b0, sem),
                    pltpu.async_copy(y1_hbm.at[s2_v.at[c]], b1, sem))

        pending = issue(0)
        for c in range(NCH):
            for d in pending:
                d.wait()
            if c + 1 < NCH:
                pending = issue(c + 1)
            a0, a1, b0, b1, _ = bufs[c % 2]
            for t in range(CH):
                w1s = w1_v[c * CH + t, :]
                w2s = w2_v[c * CH + t, :]

                def body(v, carry, t=t, w1s=w1s, w2s=w2s,
                         a0=a0, a1=a1, b0=b0, b1=b1):
                    sl = pl.ds(v * 16, 16)
                    o_v[t, sl] = (w1s * (a0[t, sl] + a1[t, sl])
                                  + w2s * (b0[t, sl] + b1[t, sl]))
                    return carry

                lax.fori_loop(0, DIM // 16, body, 0)
            pltpu.sync_copy(o_v, out_hbm.at[pl.ds(tokbase + c * CH, CH)])

    return k(y0, y1, s1_2d, s2_2d, w1e, w2e)


def kernel(hidden_states, router_w, gate_w, up_w, down_w):
    B, S, D = hidden_states.shape
    T = B * S
    x = hidden_states.reshape(T, D)

    w, e = _route(x, router_w)

    e_flat = e.reshape(-1)
    oh = (e_flat[:, None] == jnp.arange(NUM_EXPERTS)[None, :]).astype(jnp.int32)
    pos = jnp.cumsum(oh, axis=0) - oh
    pos_i = jnp.sum(pos * oh, axis=-1)
    counts = jnp.sum(oh, axis=0)
    padded = ((counts + TM - 1) // TM) * TM
    ends = jnp.cumsum(padded)
    offs = ends - padded
    slot = offs[e_flat] + pos_i

    R = T * TOP_K + NUM_EXPERTS * TM
    gidx = jnp.zeros((R,), jnp.int32).at[slot].set(
        jnp.arange(T * TOP_K, dtype=jnp.int32) // TOP_K)
    n_tiles = R // TM
    tile_expert = jnp.minimum(
        jnp.sum(jnp.arange(n_tiles)[:, None] >= (ends // TM)[None, :], axis=-1),
        NUM_EXPERTS - 1).astype(jnp.int32)

    xg = jnp.take(x.astype(jnp.bfloat16), gidx, axis=0)
    y0 = _grouped_mlp_half(xg, tile_expert, gate_w, up_w, down_w, 0)
    y1 = _grouped_mlp_half(xg, tile_expert, gate_w, up_w, down_w, 1)

    s_tok = slot.reshape(T, TOP_K)
    out = _sc_combine(
        y0, y1,
        (s_tok[:, 0] + 0).reshape(T // 8, 8),
        (s_tok[:, 1] + 0).reshape(T // 8, 8),
        jnp.broadcast_to(w[:, 0:1], (T, 16)) + 0.0,
        jnp.broadcast_to(w[:, 1:2], (T, 16)) + 0.0)
    return out.reshape(B, S, D)
